# baseline (speedup 1.0000x reference)
import jax
import jax.numpy as jnp
from jax import lax
from jax.experimental import pallas as pl
from jax.experimental.pallas import tpu as pltpu
from jax.experimental.pallas import tpu_sc as plsc

N_NODES = 10000
N_EDGES = 320000
D = 128

NC = 2
NS = 16
NW = NC * NS
C = 128
NCHUNK = 80
EDGES_PER_W = NCHUNK * C
E_PAD = NW * EDGES_PER_W
ACC_ROWS = 10240
DUMMY_ROW = 10016
ROWS_PER_TILE = ACC_ROWS // NS
NZCOPY = ROWS_PER_TILE // C
DEG_W = 16

_MESH = plsc.VectorSubcoreMesh(
    core_axis_name="c", subcore_axis_name="s", num_cores=NC, num_subcores=NS
)


def _agg_body(h_hbm, e_hbm, acc_hbm, acc_sh,
              idx0, idx1, idx2, idx3, rows0, rows1,
              isem0, isem1, isem2, isem3, gsem0, gsem1, ssem0, ssem1):
    idx = (idx0, idx1, idx2, idx3)
    rows = (rows0, rows1)
    isem = (isem0, isem1, isem2, isem3)
    gsem = (gsem0, gsem1)
    ssem = (ssem0, ssem1)
    cid = lax.axis_index("c")
    sid = lax.axis_index("s")
    wid = cid * NS + sid

    zeros16 = jnp.zeros((16,), jnp.float32)

    def _zrow(r, _):
        for c16 in range(D // 16):
            rows0[r, pl.ds(c16 * 16, 16)] = zeros16
        return 0

    lax.fori_loop(0, C, _zrow, 0)
    for t in range(NZCOPY):
        pltpu.sync_copy(rows0, acc_sh.at[pl.ds(sid * ROWS_PER_TILE + t * C, C)])

    plsc.subcore_barrier()

    for q in range(4):
        pltpu.async_copy(e_hbm.at[wid, q], idx[q], isem[q])
    pltpu.make_async_copy(e_hbm.at[wid, 0], idx[0], isem[0]).wait()
    pltpu.async_copy(h_hbm.at[idx[0].at[0]], rows0, gsem0)

    def _chunk(j, k):
        b = k % 2
        nb = 1 - b
        q = k % 4
        pq = (k + 3) % 4
        pltpu.make_async_copy(h_hbm.at[idx[q].at[0]], rows[b], gsem[b]).wait()
        pltpu.async_copy(rows[b], acc_sh.at[idx[q].at[1]], ssem[b], add=True)

        @pl.when(j >= 1)
        def _():
            pltpu.make_async_copy(
                rows[nb], acc_sh.at[idx[pq].at[1]], ssem[nb]
            ).wait()

        @pl.when(j < NCHUNK - 1)
        def _():
            nq = (k + 1) % 4
            pltpu.make_async_copy(e_hbm.at[wid, j + 1], idx[nq], isem[nq]).wait()
            pltpu.async_copy(h_hbm.at[idx[nq].at[0]], rows[nb], gsem[nb])

        @pl.when((j >= 1) & (j < NCHUNK - 3))
        def _():
            pltpu.async_copy(e_hbm.at[wid, j + 3], idx[pq], isem[pq])

    def _quad(g, x):
        for k in range(4):
            _chunk(4 * g + k, k)
        return x

    lax.fori_loop(0, NCHUNK // 4, _quad, 0)
    pltpu.make_async_copy(
        rows[(NCHUNK - 1) % 2], acc_sh.at[idx[(NCHUNK - 1) % 4].at[1]],
        ssem[(NCHUNK - 1) % 2]
    ).wait()

    plsc.subcore_barrier()

    for t in range(NZCOPY):
        r0 = sid * ROWS_PER_TILE + t * C
        pltpu.sync_copy(acc_sh.at[pl.ds(r0, C)], rows0)
        pltpu.sync_copy(rows0, acc_hbm.at[cid, pl.ds(r0, C)])


_sc_agg = pl.kernel(
    _agg_body,
    out_type=jax.ShapeDtypeStruct((NC, ACC_ROWS, D), jnp.float32),
    mesh=_MESH,
    scratch_types=[
        pltpu.VMEM_SHARED((ACC_ROWS, D), jnp.float32),
        pltpu.VMEM((2, C), jnp.int32),
        pltpu.VMEM((2, C), jnp.int32),
        pltpu.VMEM((2, C), jnp.int32),
        pltpu.VMEM((2, C), jnp.int32),
        pltpu.VMEM((C, D), jnp.float32),
        pltpu.VMEM((C, D), jnp.float32),
    ] + [pltpu.SemaphoreType.DMA] * 8,
)


def _deg_body(e_hbm, deg_hbm, deg_sh, idx0, idx1, ones_v, isem0, isem1):
    idx = (idx0, idx1)
    isem = (isem0, isem1)
    cid = lax.axis_index("c")
    sid = lax.axis_index("s")
    wid = cid * NS + sid

    zeros16 = jnp.zeros((16,), jnp.float32)

    def _z(r, _):
        ones_v[r, pl.ds(0, 16)] = zeros16
        return 0

    lax.fori_loop(0, C, _z, 0)
    for t in range(NZCOPY):
        pltpu.sync_copy(ones_v, deg_sh.at[pl.ds(sid * ROWS_PER_TILE + t * C, C)])
    ones16 = jnp.ones((16,), jnp.float32)

    def _f(r, _):
        ones_v[r, pl.ds(0, 16)] = ones16
        return 0

    lax.fori_loop(0, C, _f, 0)

    plsc.subcore_barrier()

    pltpu.async_copy(e_hbm.at[wid, 0], idx0, isem0)
    pltpu.async_copy(e_hbm.at[wid, 1], idx1, isem1)

    def _chunk(j, b):
        pltpu.make_async_copy(e_hbm.at[wid, j], idx[b], isem[b]).wait()
        pltpu.sync_copy(ones_v, deg_sh.at[idx[b].at[1]], add=True)

        @pl.when(j < NCHUNK - 2)
        def _():
            pltpu.async_copy(e_hbm.at[wid, j + 2], idx[b], isem[b])

    def _pair(g, x):
        _chunk(2 * g, 0)
        _chunk(2 * g + 1, 1)
        return x

    lax.fori_loop(0, NCHUNK // 2, _pair, 0)

    plsc.subcore_barrier()

    for t in range(NZCOPY):
        r0 = sid * ROWS_PER_TILE + t * C
        pltpu.sync_copy(deg_sh.at[pl.ds(r0, C)], ones_v)
        pltpu.sync_copy(ones_v, deg_hbm.at[cid, pl.ds(r0, C)])


_sc_deg = pl.kernel(
    _deg_body,
    out_type=jax.ShapeDtypeStruct((NC, ACC_ROWS, DEG_W), jnp.float32),
    mesh=_MESH,
    scratch_types=[
        pltpu.VMEM_SHARED((ACC_ROWS, DEG_W), jnp.float32),
        pltpu.VMEM((2, C), jnp.int32),
        pltpu.VMEM((2, C), jnp.int32),
        pltpu.VMEM((C, DEG_W), jnp.float32),
        pltpu.SemaphoreType.DMA,
        pltpu.SemaphoreType.DMA,
    ],
)


def _tc_body(h_ref, acc_ref, deg_ref, ws_ref, wa_ref, out_ref):
    deg = jnp.maximum(jnp.sum(deg_ref[...], axis=(0, 2)), 1.0)
    agg = (acc_ref[0] + acc_ref[1]) * (1.0 / deg)[:, None]
    out = jnp.dot(h_ref[...], ws_ref[...], preferred_element_type=jnp.float32)
    out += jnp.dot(agg, wa_ref[...], preferred_element_type=jnp.float32)
    out_ref[...] = jnp.maximum(out, 0.0)


_TC_R = 1000


def _tc_layer(h, acc, deg, ws_t, wa_t):
    grid = N_NODES // _TC_R
    return pl.pallas_call(
        _tc_body,
        grid=(grid,),
        in_specs=[
            pl.BlockSpec((_TC_R, D), lambda i: (i, 0)),
            pl.BlockSpec((NC, _TC_R, D), lambda i: (0, i, 0)),
            pl.BlockSpec((NC, _TC_R, DEG_W), lambda i: (0, i, 0)),
            pl.BlockSpec((D, D), lambda i: (0, 0)),
            pl.BlockSpec((D, D), lambda i: (0, 0)),
        ],
        out_specs=pl.BlockSpec((_TC_R, D), lambda i: (i, 0)),
        out_shape=jax.ShapeDtypeStruct((N_NODES, D), jnp.float32),
    )(h, acc, deg, ws_t, wa_t)


def kernel(x, edge_index, W1, W2):
    src = edge_index[0].astype(jnp.int32)
    dst = edge_index[1].astype(jnp.int32)
    pad = E_PAD - N_EDGES
    src3 = jnp.concatenate([src, jnp.zeros((pad,), jnp.int32)]).reshape(NW, NCHUNK, C)
    dst3 = jnp.concatenate(
        [dst, jnp.full((pad,), DUMMY_ROW, jnp.int32)]
    ).reshape(NW, NCHUNK, C)
    edges = jnp.stack([src3, dst3], axis=2)

    w1s_t = W1[:, :D].T
    w1a_t = W1[:, D:].T
    w2s_t = W2[:, :D].T
    w2a_t = W2[:, D:].T

    deg = _sc_deg(edges)
    acc1 = _sc_agg(x, edges)
    h1 = _tc_layer(x, acc1, deg, w1s_t, w1a_t)
    acc2 = _sc_agg(h1, edges)
    return _tc_layer(h1, acc2, deg, w2s_t, w2a_t)


# speedup vs baseline: 1.1162x; 1.1162x over previous
import jax
import jax.numpy as jnp
from jax import lax
from jax.experimental import pallas as pl
from jax.experimental.pallas import tpu as pltpu
from jax.experimental.pallas import tpu_sc as plsc

N_NODES = 10000
N_EDGES = 320000
D = 128

NC = 2
NS = 16
NW = NC * NS
C = 128
NCHUNK = 80
EDGES_PER_W = NCHUNK * C
E_PAD = NW * EDGES_PER_W
ACC_ROWS = 10240
DUMMY_ROW = 10016
ROWS_PER_TILE = ACC_ROWS // NS
NZCOPY = ROWS_PER_TILE // C
DEG_W = 16

_MESH = plsc.VectorSubcoreMesh(
    core_axis_name="c", subcore_axis_name="s", num_cores=NC, num_subcores=NS
)


def _agg_body(h_hbm, e_hbm, acc_hbm, acc_sh, idx0, idx1, rows0, rows1,
              isem0, isem1, gsem0, gsem1):
    idx = (idx0, idx1)
    rows = (rows0, rows1)
    isem = (isem0, isem1)
    gsem = (gsem0, gsem1)
    cid = lax.axis_index("c")
    sid = lax.axis_index("s")
    wid = cid * NS + sid

    zeros16 = jnp.zeros((16,), jnp.float32)

    def _zrow(r, _):
        for c16 in range(D // 16):
            rows0[r, pl.ds(c16 * 16, 16)] = zeros16
        return 0

    lax.fori_loop(0, C, _zrow, 0)
    for t in range(NZCOPY):
        pltpu.sync_copy(rows0, acc_sh.at[pl.ds(sid * ROWS_PER_TILE + t * C, C)])

    plsc.subcore_barrier()

    pltpu.async_copy(e_hbm.at[wid, 0], idx0, isem0)
    pltpu.async_copy(e_hbm.at[wid, 1], idx1, isem1)
    pltpu.make_async_copy(e_hbm.at[wid, 0], idx0, isem0).wait()
    pltpu.async_copy(h_hbm.at[idx0.at[0]], rows0, gsem0)

    def _chunk(j, b):
        nb = 1 - b
        pltpu.make_async_copy(h_hbm.at[idx[b].at[0]], rows[b], gsem[b]).wait()

        @pl.when(j < NCHUNK - 1)
        def _():
            pltpu.make_async_copy(e_hbm.at[wid, j + 1], idx[nb], isem[nb]).wait()
            pltpu.async_copy(h_hbm.at[idx[nb].at[0]], rows[nb], gsem[nb])

        pltpu.sync_copy(rows[b], acc_sh.at[idx[b].at[1]], add=True)

        @pl.when(j < NCHUNK - 2)
        def _():
            pltpu.async_copy(e_hbm.at[wid, j + 2], idx[b], isem[b])

    def _pair(g, x):
        _chunk(2 * g, 0)
        _chunk(2 * g + 1, 1)
        return x

    lax.fori_loop(0, NCHUNK // 2, _pair, 0)

    plsc.subcore_barrier()

    for t in range(NZCOPY):
        r0 = sid * ROWS_PER_TILE + t * C
        pltpu.sync_copy(acc_sh.at[pl.ds(r0, C)], rows0)
        pltpu.sync_copy(rows0, acc_hbm.at[cid, pl.ds(r0, C)])


_sc_agg = pl.kernel(
    _agg_body,
    out_type=jax.ShapeDtypeStruct((NC, ACC_ROWS, D), jnp.float32),
    mesh=_MESH,
    scratch_types=[
        pltpu.VMEM_SHARED((ACC_ROWS, D), jnp.float32),
        pltpu.VMEM((2, C), jnp.int32),
        pltpu.VMEM((2, C), jnp.int32),
        pltpu.VMEM((C, D), jnp.float32),
        pltpu.VMEM((C, D), jnp.float32),
    ] + [pltpu.SemaphoreType.DMA] * 4,
)


def _deg_body(e_hbm, deg_hbm, deg_sh, idx0, idx1, ones_v, isem0, isem1):
    idx = (idx0, idx1)
    isem = (isem0, isem1)
    cid = lax.axis_index("c")
    sid = lax.axis_index("s")
    wid = cid * NS + sid

    zeros16 = jnp.zeros((16,), jnp.float32)
    for r in range(C // 16):
        ones_v[pl.ds(16 * r, 16)] = zeros16
    for t in range(NZCOPY):
        pltpu.sync_copy(ones_v, deg_sh.at[pl.ds(sid * ROWS_PER_TILE + t * C, C)])
    ones16 = jnp.ones((16,), jnp.float32)
    for r in range(C // 16):
        ones_v[pl.ds(16 * r, 16)] = ones16

    plsc.subcore_barrier()

    pltpu.async_copy(e_hbm.at[wid, 0], idx0, isem0)
    pltpu.async_copy(e_hbm.at[wid, 1], idx1, isem1)

    def _chunk(j, b):
        pltpu.make_async_copy(e_hbm.at[wid, j], idx[b], isem[b]).wait()
        pltpu.sync_copy(ones_v, deg_sh.at[idx[b].at[1]], add=True)

        @pl.when(j < NCHUNK - 2)
        def _():
            pltpu.async_copy(e_hbm.at[wid, j + 2], idx[b], isem[b])

    def _pair(g, x):
        _chunk(2 * g, 0)
        _chunk(2 * g + 1, 1)
        return x

    lax.fori_loop(0, NCHUNK // 2, _pair, 0)

    plsc.subcore_barrier()

    for t in range(NZCOPY):
        r0 = sid * ROWS_PER_TILE + t * C
        pltpu.sync_copy(deg_sh.at[pl.ds(r0, C)], ones_v)
        pltpu.sync_copy(ones_v, deg_hbm.at[cid, pl.ds(r0, C)])


_sc_deg = pl.kernel(
    _deg_body,
    out_type=jax.ShapeDtypeStruct((NC, ACC_ROWS), jnp.float32),
    mesh=_MESH,
    scratch_types=[
        pltpu.VMEM_SHARED((ACC_ROWS,), jnp.float32),
        pltpu.VMEM((2, C), jnp.int32),
        pltpu.VMEM((2, C), jnp.int32),
        pltpu.VMEM((C,), jnp.float32),
        pltpu.SemaphoreType.DMA,
        pltpu.SemaphoreType.DMA,
    ],
)


def _tc_body(h_ref, acc_ref, deg_ref, ws_ref, wa_ref, out_ref):
    deg = jnp.maximum(jnp.sum(deg_ref[...], axis=1), 1.0)
    agg = (acc_ref[0] + acc_ref[1]) * (1.0 / deg)[:, None]
    out = jnp.dot(h_ref[...], ws_ref[...], preferred_element_type=jnp.float32)
    out += jnp.dot(agg, wa_ref[...], preferred_element_type=jnp.float32)
    out_ref[...] = jnp.maximum(out, 0.0)


_TC_R = 1000


def _tc_layer(h, acc, deg, ws_t, wa_t):
    grid = N_NODES // _TC_R
    return pl.pallas_call(
        _tc_body,
        grid=(grid,),
        in_specs=[
            pl.BlockSpec((_TC_R, D), lambda i: (i, 0)),
            pl.BlockSpec((NC, _TC_R, D), lambda i: (0, i, 0)),
            pl.BlockSpec((_TC_R, NC), lambda i: (i, 0)),
            pl.BlockSpec((D, D), lambda i: (0, 0)),
            pl.BlockSpec((D, D), lambda i: (0, 0)),
        ],
        out_specs=pl.BlockSpec((_TC_R, D), lambda i: (i, 0)),
        out_shape=jax.ShapeDtypeStruct((N_NODES, D), jnp.float32),
    )(h, acc, deg, ws_t, wa_t)


def kernel(x, edge_index, W1, W2):
    src = edge_index[0].astype(jnp.int32)
    dst = edge_index[1].astype(jnp.int32)
    pad = E_PAD - N_EDGES
    src3 = jnp.concatenate([src, jnp.zeros((pad,), jnp.int32)]).reshape(NW, NCHUNK, C)
    dst3 = jnp.concatenate(
        [dst, jnp.full((pad,), DUMMY_ROW, jnp.int32)]
    ).reshape(NW, NCHUNK, C)
    edges = jnp.stack([src3, dst3], axis=2)

    w1s_t = W1[:, :D].T
    w1a_t = W1[:, D:].T
    w2s_t = W2[:, :D].T
    w2a_t = W2[:, D:].T

    deg = _sc_deg(edges).T
    acc1 = _sc_agg(x, edges)
    h1 = _tc_layer(x, acc1, deg, w1s_t, w1a_t)
    acc2 = _sc_agg(h1, edges)
    return _tc_layer(h1, acc2, deg, w2s_t, w2a_t)
